# baseline (speedup 1.0000x reference)
import functools
import math

import jax
import jax.numpy as jnp
from jax import lax
from jax.experimental import pallas as pl
from jax.experimental.pallas import tpu as pltpu
from jax.experimental.pallas import tpu_sc as plsc

_D = 64
_SCALE = math.sqrt(_D)
_NC, _NS, _L = 2, 16, 16
_NW = _NC * _NS
_G = 128
_K = 4
_CH = _K * _G


@functools.partial(jax.jit, static_argnames=())
def _embed_flat(x3, lut):
    nw, n_g, g = x3.shape
    per_w = n_g * g
    b = nw * per_w
    n_ch = per_w // _CH

    mesh = plsc.VectorSubcoreMesh(
        core_axis_name="c", subcore_axis_name="s", num_cores=_NC, num_subcores=_NS
    )

    @functools.partial(
        pl.kernel,
        mesh=mesh,
        out_type=jax.ShapeDtypeStruct((b, _D), jnp.float32),
        compiler_params=pltpu.CompilerParams(use_tc_tiling_on_sc=False),
        scratch_types=[
            pltpu.VMEM((n_g, _G), jnp.int32),
            pltpu.VMEM((_CH, _D), jnp.float32),
            pltpu.SemaphoreType.DMA,
        ],
    )
    def body(x_hbm, lut_hbm, out_hbm, idx_v, rows_v, sem):
        wid = lax.axis_index("s") * _NC + lax.axis_index("c")
        base = wid * per_w
        pltpu.sync_copy(x_hbm.at[wid], idx_v)

        def chunk(c, carry):
            copies = [
                pltpu.async_copy(
                    lut_hbm.at[idx_v.at[c * _K + j]],
                    rows_v.at[pl.ds(j * _G, _G)],
                    sem,
                )
                for j in range(_K)
            ]
            for cp in copies:
                cp.wait()

            def scale_row(r, acc):
                for q in range(_D // _L):
                    sl = pl.ds(q * _L, _L)
                    rows_v[r, sl] = rows_v[r, sl] * _SCALE
                return acc

            lax.fori_loop(0, _CH, scale_row, 0, unroll=4)
            pltpu.sync_copy(rows_v, out_hbm.at[pl.ds(base + c * _CH, _CH)])
            return carry

        lax.fori_loop(0, n_ch, chunk, 0)

    return body(x3, lut)


def kernel(x, lut):
    bs, t = x.shape
    b = bs * t
    per_w = b // _NW
    n_g = per_w // _G
    x3 = x.reshape(_NW, n_g, _G).astype(jnp.int32)
    out = _embed_flat(x3, lut)
    return out.reshape(bs, t, _D)
